# Initial kernel scaffold; baseline (speedup 1.0000x reference)
import jax
import jax.numpy as jnp
from jax.experimental import pallas as pl


def kernel(x, h0, W_ih, b_ih, W_hh, b_hh, W_out, b_out):
    raise NotImplementedError("write your pallas kernel here")


# speedup vs baseline: 12.7911x; 12.7911x over previous
from functools import partial

import jax
import jax.numpy as jnp
from jax.experimental import pallas as pl
from jax.experimental.pallas import tpu as pltpu


def _rnn_body(x_ref, h0_ref, wih_ref, whh_ref, wout_ref, bias_ref, bout_ref,
              y_ref, h_s, hall_s, *, TB, BB):
    j = pl.program_id(1)
    I = x_ref.shape[-1]
    H = whh_ref.shape[0]
    O = wout_ref.shape[-1]

    xp = jnp.dot(x_ref[...].reshape(TB * BB, I), wih_ref[...],
                 preferred_element_type=jnp.float32) + bias_ref[...]

    h = jnp.where(j == 0, h0_ref[...], h_s[...])
    whh = whh_ref[...]
    for i in range(TB):
        h = jnp.tanh(xp[i * BB:(i + 1) * BB, :]
                     + jnp.dot(h, whh, preferred_element_type=jnp.float32))
        hall_s[i] = h
    h_s[...] = h

    y = jnp.dot(hall_s[...].reshape(TB * BB, H), wout_ref[...],
                preferred_element_type=jnp.float32) + bout_ref[...]
    y_ref[...] = y.reshape(TB, BB, O)


def kernel(x, h0, W_ih, b_ih, W_hh, b_hh, W_out, b_out):
    T, B, I = x.shape
    H = W_ih.shape[0]
    O = W_out.shape[0]
    TB = 16
    NB = 2
    BB = B // NB
    NT = T // TB

    bias = (b_ih + b_hh).reshape(1, H)
    bout = b_out.reshape(1, O)

    return pl.pallas_call(
        partial(_rnn_body, TB=TB, BB=BB),
        grid=(NB, NT),
        in_specs=[
            pl.BlockSpec((TB, BB, I), lambda b, t: (t, b, 0)),
            pl.BlockSpec((BB, H), lambda b, t: (b, 0)),
            pl.BlockSpec((I, H), lambda b, t: (0, 0)),
            pl.BlockSpec((H, H), lambda b, t: (0, 0)),
            pl.BlockSpec((H, O), lambda b, t: (0, 0)),
            pl.BlockSpec((1, H), lambda b, t: (0, 0)),
            pl.BlockSpec((1, O), lambda b, t: (0, 0)),
        ],
        out_specs=pl.BlockSpec((TB, BB, O), lambda b, t: (t, b, 0)),
        out_shape=jax.ShapeDtypeStruct((T, B, O), jnp.float32),
        scratch_shapes=[
            pltpu.VMEM((BB, H), jnp.float32),
            pltpu.VMEM((TB, BB, H), jnp.float32),
        ],
        compiler_params=pltpu.CompilerParams(
            dimension_semantics=("parallel", "arbitrary"),
        ),
        name="elman_rnn_fused",
    )(x, h0, W_ih.T, W_hh.T, W_out.T, bias, bout)


# speedup vs baseline: 21.0878x; 1.6486x over previous
from functools import partial

import jax
import jax.numpy as jnp
from jax.experimental import pallas as pl
from jax.experimental.pallas import tpu as pltpu


def _rnn_body(x_ref, h0_ref, wih_ref, whh_ref, wout_ref, bias_ref, bout_ref,
              y_ref, h_s, xp_s, hall_s, *, TB, BB, NT):
    j = pl.program_id(0)
    I = x_ref.shape[-1]
    H = whh_ref.shape[0]
    O = wout_ref.shape[-1]
    p_cur = jax.lax.rem(j, 2)
    p_prev = jax.lax.rem(j + 1, 2)

    hall = hall_s[p_cur].reshape(TB * BB, H)
    y = jnp.dot(hall, wout_ref[...],
                preferred_element_type=jnp.float32) + bout_ref[...]
    y_ref[...] = y.reshape(TB, BB, O)

    h = jnp.where(j == 1, h0_ref[...], h_s[...])
    whh = whh_ref[...]
    for i in range(TB):
        z = xp_s[p_prev, i] + jnp.dot(h, whh,
                                      preferred_element_type=jnp.float32)
        h = jnp.tanh(z)
        hall_s[p_prev, i] = h
    h_s[...] = h

    xp = jnp.dot(x_ref[...].reshape(TB * BB, I), wih_ref[...],
                 preferred_element_type=jnp.float32) + bias_ref[...]
    xp3 = xp.reshape(TB, BB, H)
    half = TB // 2
    xp_s[p_cur, 0:half] = xp3[0:half]
    xp_s[p_cur, half:TB] = xp3[half:TB]


def kernel(x, h0, W_ih, b_ih, W_hh, b_hh, W_out, b_out):
    T, B, I = x.shape
    H = W_ih.shape[0]
    O = W_out.shape[0]
    TB = 16
    BB = B
    NT = T // TB

    bias = (b_ih + b_hh).reshape(1, H)
    bout = b_out.reshape(1, O)

    return pl.pallas_call(
        partial(_rnn_body, TB=TB, BB=BB, NT=NT),
        grid=(NT + 2,),
        in_specs=[
            pl.BlockSpec((TB, BB, I), lambda j: (jnp.minimum(j, NT - 1), 0, 0)),
            pl.BlockSpec((BB, H), lambda j: (0, 0)),
            pl.BlockSpec((I, H), lambda j: (0, 0)),
            pl.BlockSpec((H, H), lambda j: (0, 0)),
            pl.BlockSpec((H, O), lambda j: (0, 0)),
            pl.BlockSpec((1, H), lambda j: (0, 0)),
            pl.BlockSpec((1, O), lambda j: (0, 0)),
        ],
        out_specs=pl.BlockSpec(
            (TB, BB, O), lambda j: (jnp.maximum(j - 2, 0), 0, 0)),
        out_shape=jax.ShapeDtypeStruct((T, B, O), jnp.float32),
        scratch_shapes=[
            pltpu.VMEM((BB, H), jnp.float32),
            pltpu.VMEM((2, TB, BB, H), jnp.float32),
            pltpu.VMEM((2, TB, BB, H), jnp.float32),
        ],
        compiler_params=pltpu.CompilerParams(
            dimension_semantics=("arbitrary",),
        ),
        name="elman_rnn_skewed",
    )(x, h0, W_ih.T, W_hh.T, W_out.T, bias, bout)


# speedup vs baseline: 23.4384x; 1.1115x over previous
from functools import partial

import jax
import jax.numpy as jnp
from jax.experimental import pallas as pl
from jax.experimental.pallas import tpu as pltpu


def _rnn_body(x_ref, h0_ref, wih_ref, whh_ref, wout_ref, bias_ref, bout_ref,
              y_ref, h_s, xp_e, xp_o, hall_e, hall_o, *, TB, BB, NT2):
    k = pl.program_id(0)
    I = x_ref.shape[-1]
    H = whh_ref.shape[0]
    O = wout_ref.shape[-1]
    CH = 4
    TC = TB // CH
    whh = whh_ref[...]

    def a_chunk(c, x_off, xp_dst):
        xin = x_ref[x_off + c * TC:x_off + (c + 1) * TC].reshape(TC * BB, I)
        xp = jnp.dot(xin, wih_ref[...],
                     preferred_element_type=jnp.float32) + bias_ref[...]
        xp_dst[c * TC:(c + 1) * TC] = xp.reshape(TC, BB, H)

    def c_chunk(c, hall_src, y_off):
        hall = hall_src[c * TC:(c + 1) * TC].reshape(TC * BB, H)
        y = jnp.dot(hall, wout_ref[...],
                    preferred_element_type=jnp.float32) + bout_ref[...]
        y_ref[y_off + c * TC:y_off + (c + 1) * TC] = y.reshape(TC, BB, O)

    h = h_s[...]
    for i in range(TB):
        z = xp_o[i] + jnp.dot(h, whh, preferred_element_type=jnp.float32)
        if i % 4 == 1:
            a_chunk(i // 4, 0, xp_e)
        elif i % 4 == 3:
            c_chunk(i // 4, hall_e, 0)
        h = jnp.tanh(z)
        hall_o[i] = h

    h = jnp.where(k == 0, h0_ref[...], h)

    for i in range(TB):
        z = xp_e[i] + jnp.dot(h, whh, preferred_element_type=jnp.float32)
        if i % 4 == 1:
            a_chunk(i // 4, TB, xp_o)
        elif i % 4 == 3:
            c_chunk(i // 4, hall_o, TB)
        h = jnp.tanh(z)
        hall_e[i] = h
    h_s[...] = h


def kernel(x, h0, W_ih, b_ih, W_hh, b_hh, W_out, b_out):
    T, B, I = x.shape
    H = W_ih.shape[0]
    O = W_out.shape[0]
    TB = 16
    BB = B
    NT2 = T // (2 * TB)

    bias = (b_ih + b_hh).reshape(1, H)
    bout = b_out.reshape(1, O)

    return pl.pallas_call(
        partial(_rnn_body, TB=TB, BB=BB, NT2=NT2),
        grid=(NT2 + 1,),
        in_specs=[
            pl.BlockSpec((2 * TB, BB, I),
                         lambda k: (jnp.minimum(k, NT2 - 1), 0, 0)),
            pl.BlockSpec((BB, H), lambda k: (0, 0)),
            pl.BlockSpec((I, H), lambda k: (0, 0)),
            pl.BlockSpec((H, H), lambda k: (0, 0)),
            pl.BlockSpec((H, O), lambda k: (0, 0)),
            pl.BlockSpec((1, H), lambda k: (0, 0)),
            pl.BlockSpec((1, O), lambda k: (0, 0)),
        ],
        out_specs=pl.BlockSpec(
            (2 * TB, BB, O), lambda k: (jnp.maximum(k - 1, 0), 0, 0)),
        out_shape=jax.ShapeDtypeStruct((T, B, O), jnp.float32),
        scratch_shapes=[
            pltpu.VMEM((BB, H), jnp.float32),
            pltpu.VMEM((TB, BB, H), jnp.float32),
            pltpu.VMEM((TB, BB, H), jnp.float32),
            pltpu.VMEM((TB, BB, H), jnp.float32),
            pltpu.VMEM((TB, BB, H), jnp.float32),
        ],
        compiler_params=pltpu.CompilerParams(
            dimension_semantics=("arbitrary",),
            vmem_limit_bytes=50 * 1024 * 1024,
        ),
        name="elman_rnn_paired",
    )(x, h0, W_ih.T, W_hh.T, W_out.T, bias, bout)


# speedup vs baseline: 26.7691x; 1.1421x over previous
from functools import partial

import jax
import jax.numpy as jnp
from jax.experimental import pallas as pl
from jax.experimental.pallas import tpu as pltpu


def _rnn_body(x_ref, h0_ref, wih_ref, whh_ref, wout_ref, bias_ref, bout_ref,
              y_ref, h_s, xp_e, xp_o, hall_e, hall_o, *, TB, BB, NT2):
    k = pl.program_id(0)
    I = x_ref.shape[-1]
    H = whh_ref.shape[0]
    O = wout_ref.shape[-1]
    CH = 8
    TC = TB // CH
    SP = TB // CH
    a_pos = {c * SP: c for c in range(CH)}
    c_pos = {c * SP + SP // 2: c for c in range(CH)}
    whh = whh_ref[...]

    def a_chunk(c, x_off, xp_dst):
        xin = x_ref[x_off + c * TC:x_off + (c + 1) * TC].reshape(TC * BB, I)
        xp = jnp.dot(xin, wih_ref[...],
                     preferred_element_type=jnp.float32) + bias_ref[...]
        xp_dst[c * TC:(c + 1) * TC] = xp.reshape(TC, BB, H)

    def c_chunk(c, hall_src, y_off):
        hall = hall_src[c * TC:(c + 1) * TC].reshape(TC * BB, H)
        y = jnp.dot(hall, wout_ref[...],
                    preferred_element_type=jnp.float32) + bout_ref[...]
        y_ref[y_off + c * TC:y_off + (c + 1) * TC] = y.reshape(TC, BB, O)

    h = h_s[...]
    for i in range(TB):
        z = xp_o[i] + jnp.dot(h, whh, preferred_element_type=jnp.float32)
        if i in a_pos:
            a_chunk(a_pos[i], 0, xp_e)
        if i in c_pos:
            c_chunk(c_pos[i], hall_e, 0)
        h = jnp.tanh(z)
        hall_o[i] = h

    h = jnp.where(k == 0, h0_ref[...], h)

    for i in range(TB):
        z = xp_e[i] + jnp.dot(h, whh, preferred_element_type=jnp.float32)
        if i in a_pos:
            a_chunk(a_pos[i], TB, xp_o)
        if i in c_pos:
            c_chunk(c_pos[i], hall_o, TB)
        h = jnp.tanh(z)
        hall_e[i] = h
    h_s[...] = h


def kernel(x, h0, W_ih, b_ih, W_hh, b_hh, W_out, b_out):
    T, B, I = x.shape
    H = W_ih.shape[0]
    O = W_out.shape[0]
    TB = 16
    BB = B
    NT2 = T // (2 * TB)

    bias = (b_ih + b_hh).reshape(1, H)
    bout = b_out.reshape(1, O)

    return pl.pallas_call(
        partial(_rnn_body, TB=TB, BB=BB, NT2=NT2),
        grid=(NT2 + 1,),
        in_specs=[
            pl.BlockSpec((2 * TB, BB, I),
                         lambda k: (jnp.minimum(k, NT2 - 1), 0, 0)),
            pl.BlockSpec((BB, H), lambda k: (0, 0)),
            pl.BlockSpec((I, H), lambda k: (0, 0)),
            pl.BlockSpec((H, H), lambda k: (0, 0)),
            pl.BlockSpec((H, O), lambda k: (0, 0)),
            pl.BlockSpec((1, H), lambda k: (0, 0)),
            pl.BlockSpec((1, O), lambda k: (0, 0)),
        ],
        out_specs=pl.BlockSpec(
            (2 * TB, BB, O), lambda k: (jnp.maximum(k - 1, 0), 0, 0)),
        out_shape=jax.ShapeDtypeStruct((T, B, O), jnp.float32),
        scratch_shapes=[
            pltpu.VMEM((BB, H), jnp.float32),
            pltpu.VMEM((TB, BB, H), jnp.float32),
            pltpu.VMEM((TB, BB, H), jnp.float32),
            pltpu.VMEM((TB, BB, H), jnp.float32),
            pltpu.VMEM((TB, BB, H), jnp.float32),
        ],
        compiler_params=pltpu.CompilerParams(
            dimension_semantics=("arbitrary",),
            vmem_limit_bytes=50 * 1024 * 1024,
        ),
        name="elman_rnn_paired",
    )(x, h0, W_ih.T, W_hh.T, W_out.T, bias, bout)
